# Initial kernel scaffold; baseline (speedup 1.0000x reference)
import jax
import jax.numpy as jnp
from jax.experimental import pallas as pl


def kernel(cls0, cls1, cls2, cnt0, cnt1, cnt2, reg0, reg1, reg2):
    raise NotImplementedError("write your pallas kernel here")


# speedup vs baseline: 140.5179x; 140.5179x over previous
import functools

import jax
import jax.numpy as jnp
import numpy as np
from jax.experimental import pallas as pl

_STRIDES = (8, 16, 32)
_SIZES = ((64, 64), (32, 32), (16, 16))
_N = 5376
_NPAD = 6144
_ROWS = 48
_PRE_NMS_K = 1000
_MAX_DET = 100
_IOU_THR = 0.5
_SCORE_THR = 0.05


def _location_consts():
    cxs, cys, svs = [], [], []
    for (h, w), s in zip(_SIZES, _STRIDES):
        ys = (np.arange(h, dtype=np.float32) + 0.5) * s
        xs = (np.arange(w, dtype=np.float32) + 0.5) * s
        cy, cx = np.meshgrid(ys, xs, indexing="ij")
        cxs.append(cx.reshape(-1))
        cys.append(cy.reshape(-1))
        svs.append(np.full(h * w, s, dtype=np.float32))
    cx = np.concatenate(cxs)
    cy = np.concatenate(cys)
    sv = np.concatenate(svs)
    pad = _NPAD - cx.shape[0]
    cx = np.pad(cx, (0, pad)).reshape(_ROWS, 128)
    cy = np.pad(cy, (0, pad)).reshape(_ROWS, 128)
    sv = np.pad(sv, (0, pad), constant_values=1.0).reshape(_ROWS, 128)
    return cx, cy, sv


_CX, _CY, _SV = _location_consts()


def _body(cls_ref, reg_ref, cx_ref, cy_ref, sv_ref, out_ref):
    p = jax.nn.sigmoid(cls_ref[0])

    score = p[0]
    kind = jnp.zeros((_ROWS, 128), jnp.float32)
    for c in range(1, 80):
        v = p[c]
        gt = v > score
        score = jnp.where(gt, v, score)
        kind = jnp.where(gt, jnp.float32(c), kind)

    cx = cx_ref[...]
    cy = cy_ref[...]
    sv = sv_ref[...]
    ltrb = jnp.exp(reg_ref[0]) * sv[None, :, :]
    x1 = cx - ltrb[0]
    y1 = cy - ltrb[1]
    x2 = cx + ltrb[2]
    y2 = cy + ltrb[3]
    areas = jnp.maximum(x2 - x1, 0.0) * jnp.maximum(y2 - y1, 0.0)

    s_pre = jnp.where(score > _SCORE_THR, score, -2.0)

    def bis(_, lohi):
        lo, hi = lohi
        mid = lo + (hi - lo + 1) // 2
        midf = jax.lax.bitcast_convert_type(jnp.full((_ROWS, 128), mid, jnp.int32), jnp.float32)
        c = jnp.sum((s_pre >= midf).astype(jnp.int32))
        ok = c >= _PRE_NMS_K
        return (jnp.where(ok, mid, lo), jnp.where(ok, hi, mid - 1))

    lo, _ = jax.lax.fori_loop(0, 31, bis, (jnp.int32(0), jnp.int32(0x3F800000)))
    tf = jax.lax.bitcast_convert_type(jnp.full((_ROWS, 128), lo, jnp.int32), jnp.float32)
    s0 = jnp.where(s_pre >= tf, s_pre, -2.0)

    li = (jax.lax.broadcasted_iota(jnp.int32, (_ROWS, 128), 0) * 128
          + jax.lax.broadcasted_iota(jnp.int32, (_ROWS, 128), 1))
    lane = jax.lax.broadcasted_iota(jnp.int32, (1, 128), 1)
    row_iota = jax.lax.broadcasted_iota(jnp.int32, (_MAX_DET, 1), 0)
    NEG = jnp.float32(-3.4e38)

    def step(_, st):
        s, out, cnt = st
        m = jnp.max(s)
        take = m > _SCORE_THR
        idx = jnp.min(jnp.where(s == m, li, jnp.int32(_NPAD)))
        sel = li == idx
        bx1 = jnp.max(jnp.where(sel, x1, NEG))
        by1 = jnp.max(jnp.where(sel, y1, NEG))
        bx2 = jnp.max(jnp.where(sel, x2, NEG))
        by2 = jnp.max(jnp.where(sel, y2, NEG))
        ba = jnp.max(jnp.where(sel, areas, NEG))
        bk = jnp.max(jnp.where(sel, kind, NEG))
        xx1 = jnp.maximum(bx1, x1)
        yy1 = jnp.maximum(by1, y1)
        xx2 = jnp.minimum(bx2, x2)
        yy2 = jnp.minimum(by2, y2)
        inter = jnp.maximum(xx2 - xx1, 0.0) * jnp.maximum(yy2 - yy1, 0.0)
        iou = inter / (ba + areas - inter + 1e-9)
        s = jnp.where(jnp.logical_and(take, iou > _IOU_THR), -2.0, s)
        vals = jnp.where(lane == 0, bx1,
               jnp.where(lane == 1, by1,
               jnp.where(lane == 2, bx2,
               jnp.where(lane == 3, by2,
               jnp.where(lane == 4, bk,
               jnp.where(lane == 5, m, 0.0))))))
        mask = jnp.logical_and(row_iota == cnt, take)
        out = jnp.where(mask, jnp.broadcast_to(vals, (_MAX_DET, 128)), out)
        cnt = cnt + take.astype(jnp.int32)
        return (s, out, cnt)

    init = (s0, jnp.zeros((_MAX_DET, 128), jnp.float32), jnp.int32(0))
    _, out, _ = jax.lax.fori_loop(0, _MAX_DET, step, init)
    out_ref[0] = out


@functools.partial(jax.jit, static_argnames=())
def kernel(cls0, cls1, cls2, cnt0, cnt1, cnt2, reg0, reg1, reg2):
    del cnt0, cnt1, cnt2
    B = cls0.shape[0]
    cls = jnp.concatenate([c.reshape(B, 80, -1) for c in (cls0, cls1, cls2)], axis=2)
    reg = jnp.concatenate([r.reshape(B, 4, -1) for r in (reg0, reg1, reg2)], axis=2)
    cls = jnp.pad(cls, ((0, 0), (0, 0), (0, _NPAD - _N)), constant_values=-30.0)
    reg = jnp.pad(reg, ((0, 0), (0, 0), (0, _NPAD - _N)))
    cls = cls.reshape(B, 80, _ROWS, 128)
    reg = reg.reshape(B, 4, _ROWS, 128)

    out = pl.pallas_call(
        _body,
        grid=(B,),
        in_specs=[
            pl.BlockSpec((1, 80, _ROWS, 128), lambda i: (i, 0, 0, 0)),
            pl.BlockSpec((1, 4, _ROWS, 128), lambda i: (i, 0, 0, 0)),
            pl.BlockSpec((_ROWS, 128), lambda i: (0, 0)),
            pl.BlockSpec((_ROWS, 128), lambda i: (0, 0)),
            pl.BlockSpec((_ROWS, 128), lambda i: (0, 0)),
        ],
        out_specs=pl.BlockSpec((1, _MAX_DET, 128), lambda i: (i, 0, 0)),
        out_shape=jax.ShapeDtypeStruct((B, _MAX_DET, 128), jnp.float32),
    )(cls, reg, jnp.asarray(_CX), jnp.asarray(_CY), jnp.asarray(_SV))
    return out[:, :, :6]


# speedup vs baseline: 157.5191x; 1.1210x over previous
import functools

import jax
import jax.numpy as jnp
import numpy as np
from jax import lax
from jax.experimental import pallas as pl
from jax.experimental.pallas import tpu as pltpu
from jax.experimental.pallas import tpu_sc as plsc

_STRIDES = (8, 16, 32)
_SIZES = ((64, 64), (32, 32), (16, 16))
_N = 5376
_NPAD = 6144
_ROWS = 48
_NCHUNK = _NPAD // 16
_PRE_NMS_K = 1000
_MAX_DET = 100
_IOU_THR = 0.5
_SCORE_THR = 0.05
_B = 4


def _location_consts():
    cxs, cys, svs = [], [], []
    for (h, w), s in zip(_SIZES, _STRIDES):
        ys = (np.arange(h, dtype=np.float32) + 0.5) * s
        xs = (np.arange(w, dtype=np.float32) + 0.5) * s
        cy, cx = np.meshgrid(ys, xs, indexing="ij")
        cxs.append(cx.reshape(-1))
        cys.append(cy.reshape(-1))
        svs.append(np.full(h * w, s, dtype=np.float32))
    cx = np.concatenate(cxs)
    cy = np.concatenate(cys)
    sv = np.concatenate(svs)
    pad = _NPAD - cx.shape[0]
    cx = np.pad(cx, (0, pad)).reshape(_ROWS, 128)
    cy = np.pad(cy, (0, pad)).reshape(_ROWS, 128)
    sv = np.pad(sv, (0, pad), constant_values=1.0).reshape(_ROWS, 128)
    return cx, cy, sv


_CX, _CY, _SV = _location_consts()


def _decode_body(cls_ref, reg_ref, cx_ref, cy_ref, sv_ref, out_ref):
    p = jax.nn.sigmoid(cls_ref[...])

    score = p[:, 0]
    kind = jnp.zeros((_B, _ROWS, 128), jnp.float32)
    for c in range(1, 80):
        v = p[:, c]
        gt = v > score
        score = jnp.where(gt, v, score)
        kind = jnp.where(gt, jnp.float32(c), kind)

    cx = cx_ref[...][None]
    cy = cy_ref[...][None]
    sv = sv_ref[...][None]
    ltrb = jnp.exp(reg_ref[...]) * sv[:, None]
    x1 = cx - ltrb[:, 0]
    y1 = cy - ltrb[:, 1]
    x2 = cx + ltrb[:, 2]
    y2 = cy + ltrb[:, 3]
    areas = jnp.maximum(x2 - x1, 0.0) * jnp.maximum(y2 - y1, 0.0)

    s_pre = jnp.where(score > _SCORE_THR, score, -2.0)

    def bis(_, lohi):
        lo, hi = lohi
        mid = lo + (hi - lo + 1) // 2
        midf = lax.bitcast_convert_type(
            jnp.broadcast_to(mid, (_B, _ROWS, 128)), jnp.float32)
        c = jnp.sum((s_pre >= midf).astype(jnp.int32), axis=(1, 2), keepdims=True)
        ok = c >= _PRE_NMS_K
        return (jnp.where(ok, mid, lo), jnp.where(ok, hi, mid - 1))

    lo0 = jnp.zeros((_B, 1, 1), jnp.int32)
    hi0 = jnp.full((_B, 1, 1), 0x3F800000, jnp.int32)
    lo, _ = lax.fori_loop(0, 31, bis, (lo0, hi0))
    tf = lax.bitcast_convert_type(jnp.broadcast_to(lo, (_B, _ROWS, 128)), jnp.float32)
    s0 = jnp.where(s_pre >= tf, s_pre, -2.0)

    out_ref[:, 0] = s0
    out_ref[:, 1] = x1
    out_ref[:, 2] = y1
    out_ref[:, 3] = x2
    out_ref[:, 4] = y2
    out_ref[:, 5] = areas
    out_ref[:, 6] = kind
    out_ref[:, 7] = jnp.zeros((_B, _ROWS, 128), jnp.float32)


def _tc_decode(cls, reg):
    return pl.pallas_call(
        _decode_body,
        in_specs=[
            pl.BlockSpec((_B, 80, _ROWS, 128), lambda: (0, 0, 0, 0)),
            pl.BlockSpec((_B, 4, _ROWS, 128), lambda: (0, 0, 0, 0)),
            pl.BlockSpec((_ROWS, 128), lambda: (0, 0)),
            pl.BlockSpec((_ROWS, 128), lambda: (0, 0)),
            pl.BlockSpec((_ROWS, 128), lambda: (0, 0)),
        ],
        out_specs=pl.BlockSpec((_B, 8, _ROWS, 128), lambda: (0, 0, 0, 0)),
        out_shape=jax.ShapeDtypeStruct((_B, 8, _ROWS, 128), jnp.float32),
    )(cls, reg, jnp.asarray(_CX), jnp.asarray(_CY), jnp.asarray(_SV))


def _sc_nms(data):
    mesh = plsc.VectorSubcoreMesh(core_axis_name="c", subcore_axis_name="s")

    @functools.partial(
        pl.kernel,
        mesh=mesh,
        compiler_params=pltpu.CompilerParams(needs_layout_passes=False),
        out_type=jax.ShapeDtypeStruct((_B, _MAX_DET, 16), jnp.float32),
        scratch_types=[
            pltpu.VMEM((8, _NPAD), jnp.float32),
            pltpu.VMEM((_MAX_DET, 16), jnp.float32),
        ],
    )
    def nms_kernel(data_hbm, out_hbm, data_v, out_v):
        w = lax.axis_index("s") * 2 + lax.axis_index("c")

        @pl.when(w < _B)
        def _():
            pltpu.sync_copy(data_hbm.at[w], data_v)
            lane = lax.iota(jnp.int32, 16)
            zero16 = jnp.zeros((16,), jnp.float32)

            def zi(i, c):
                out_v[i] = zero16
                return c

            lax.fori_loop(0, _MAX_DET, zi, 0)

            NEG = jnp.float32(-3.4e38)
            BIGI = jnp.int32(2 ** 30)

            def amax_chunk(i, st):
                bv, bl = st
                v = data_v[0, pl.ds(i * 16, 16)]
                liv = i * 16 + lane
                gt = v > bv
                bl = jnp.where(gt, liv, bl)
                bv = jnp.where(gt, v, bv)
                return bv, bl

            bv, bl = lax.fori_loop(
                0, _NCHUNK, amax_chunk,
                (jnp.full((16,), NEG), jnp.zeros((16,), jnp.int32)))
            m0 = jnp.max(bv)
            idx0 = jnp.min(jnp.where(bv == m0, bl, BIGI))

            def step_body(st):
                cnt, m, idx = st
                rowv = jnp.minimum(lane, 7)
                idxv = jnp.full((16,), idx, jnp.int32)
                g = plsc.load_gather(data_v, [rowv, idxv])

                def ext(r):
                    return jnp.max(jnp.where(lane == r, g, NEG))

                bx1 = ext(1)
                by1 = ext(2)
                bx2 = ext(3)
                by2 = ext(4)
                ba = ext(5)
                bk = ext(6)
                row = jnp.where(lane == 0, bx1,
                      jnp.where(lane == 1, by1,
                      jnp.where(lane == 2, bx2,
                      jnp.where(lane == 3, by2,
                      jnp.where(lane == 4, bk,
                      jnp.where(lane == 5, m, 0.0))))))
                out_v[cnt] = row

                def sup_chunk(i, st2):
                    bv2, bl2 = st2
                    base = i * 16
                    s = data_v[0, pl.ds(base, 16)]
                    x1 = data_v[1, pl.ds(base, 16)]
                    y1 = data_v[2, pl.ds(base, 16)]
                    x2 = data_v[3, pl.ds(base, 16)]
                    y2 = data_v[4, pl.ds(base, 16)]
                    ar = data_v[5, pl.ds(base, 16)]
                    xx1 = jnp.maximum(bx1, x1)
                    yy1 = jnp.maximum(by1, y1)
                    xx2 = jnp.minimum(bx2, x2)
                    yy2 = jnp.minimum(by2, y2)
                    inter = jnp.maximum(xx2 - xx1, 0.0) * jnp.maximum(yy2 - yy1, 0.0)
                    iou = inter / (ba + ar - inter + 1e-9)
                    s = jnp.where(iou > _IOU_THR, -2.0, s)
                    data_v[0, pl.ds(base, 16)] = s
                    liv = base + lane
                    gt = s > bv2
                    bl2 = jnp.where(gt, liv, bl2)
                    bv2 = jnp.where(gt, s, bv2)
                    return bv2, bl2

                bv2, bl2 = lax.fori_loop(
                    0, _NCHUNK, sup_chunk,
                    (jnp.full((16,), NEG), jnp.zeros((16,), jnp.int32)))
                m2 = jnp.max(bv2)
                idx2 = jnp.min(jnp.where(bv2 == m2, bl2, BIGI))
                return cnt + 1, m2, idx2

            def step(i, st):
                return lax.cond(st[1] > _SCORE_THR, step_body, lambda s: s, st)

            lax.fori_loop(0, _MAX_DET, step, (jnp.int32(0), m0, idx0))
            pltpu.sync_copy(out_v, out_hbm.at[w])

    return nms_kernel(data)


def kernel(cls0, cls1, cls2, cnt0, cnt1, cnt2, reg0, reg1, reg2):
    del cnt0, cnt1, cnt2
    B = cls0.shape[0]
    cls = jnp.concatenate([c.reshape(B, 80, -1) for c in (cls0, cls1, cls2)], axis=2)
    reg = jnp.concatenate([r.reshape(B, 4, -1) for r in (reg0, reg1, reg2)], axis=2)
    cls = jnp.pad(cls, ((0, 0), (0, 0), (0, _NPAD - _N)), constant_values=-30.0)
    reg = jnp.pad(reg, ((0, 0), (0, 0), (0, _NPAD - _N)))
    cls = cls.reshape(B, 80, _ROWS, 128)
    reg = reg.reshape(B, 4, _ROWS, 128)

    data = _tc_decode(cls, reg).reshape(B, 8, _NPAD)
    out = _sc_nms(data)
    return out[:, :, :6]


# speedup vs baseline: 306.8404x; 1.9480x over previous
import functools

import jax
import jax.numpy as jnp
import numpy as np
from jax import lax
from jax.experimental import pallas as pl
from jax.experimental.pallas import tpu as pltpu
from jax.experimental.pallas import tpu_sc as plsc

_STRIDES = (8, 16, 32)
_SIZES = ((64, 64), (32, 32), (16, 16))
_N = 5376
_NPAD = 6144
_ROWS = 48
_NCHUNK = _NPAD // 16
_PRE_NMS_K = 1000
_MAX_DET = 100
_IOU_THR = 0.5
_SCORE_THR = 0.05
_B = 4
_CAP = 1024
_CLEN = _CAP + 32


def _location_consts():
    cxs, cys, svs = [], [], []
    for (h, w), s in zip(_SIZES, _STRIDES):
        ys = (np.arange(h, dtype=np.float32) + 0.5) * s
        xs = (np.arange(w, dtype=np.float32) + 0.5) * s
        cy, cx = np.meshgrid(ys, xs, indexing="ij")
        cxs.append(cx.reshape(-1))
        cys.append(cy.reshape(-1))
        svs.append(np.full(h * w, s, dtype=np.float32))
    cx = np.concatenate(cxs)
    cy = np.concatenate(cys)
    sv = np.concatenate(svs)
    pad = _NPAD - cx.shape[0]
    cx = np.pad(cx, (0, pad)).reshape(_ROWS, 128)
    cy = np.pad(cy, (0, pad)).reshape(_ROWS, 128)
    sv = np.pad(sv, (0, pad), constant_values=1.0).reshape(_ROWS, 128)
    return cx, cy, sv


_CX, _CY, _SV = _location_consts()


def _decode_body(cls_ref, reg_ref, cx_ref, cy_ref, sv_ref, out_ref):
    p = jax.nn.sigmoid(cls_ref[...])

    score = p[:, 0]
    kind = jnp.zeros((_B, _ROWS, 128), jnp.float32)
    for c in range(1, 80):
        v = p[:, c]
        gt = v > score
        score = jnp.where(gt, v, score)
        kind = jnp.where(gt, jnp.float32(c), kind)

    cx = cx_ref[...][None]
    cy = cy_ref[...][None]
    sv = sv_ref[...][None]
    ltrb = jnp.exp(reg_ref[...]) * sv[:, None]
    x1 = cx - ltrb[:, 0]
    y1 = cy - ltrb[:, 1]
    x2 = cx + ltrb[:, 2]
    y2 = cy + ltrb[:, 3]
    areas = jnp.maximum(x2 - x1, 0.0) * jnp.maximum(y2 - y1, 0.0)

    s_pre = jnp.where(score > _SCORE_THR, score, -2.0)

    def bis(_, lohi):
        lo, hi = lohi
        mid = lo + (hi - lo + 1) // 2
        midf = lax.bitcast_convert_type(
            jnp.broadcast_to(mid, (_B, _ROWS, 128)), jnp.float32)
        c = jnp.sum((s_pre >= midf).astype(jnp.int32), axis=(1, 2), keepdims=True)
        ok = c >= _PRE_NMS_K
        return (jnp.where(ok, mid, lo), jnp.where(ok, hi, mid - 1))

    lo0 = jnp.zeros((_B, 1, 1), jnp.int32)
    hi0 = jnp.full((_B, 1, 1), 0x3F800000, jnp.int32)
    lo, _ = lax.fori_loop(0, 31, bis, (lo0, hi0))
    tf = lax.bitcast_convert_type(jnp.broadcast_to(lo, (_B, _ROWS, 128)), jnp.float32)
    s0 = jnp.where(s_pre >= tf, s_pre, -2.0)

    out_ref[:, 0] = s0
    out_ref[:, 1] = x1
    out_ref[:, 2] = y1
    out_ref[:, 3] = x2
    out_ref[:, 4] = y2
    out_ref[:, 5] = areas
    out_ref[:, 6] = kind
    out_ref[:, 7] = jnp.zeros((_B, _ROWS, 128), jnp.float32)


def _tc_decode(cls, reg):
    return pl.pallas_call(
        _decode_body,
        in_specs=[
            pl.BlockSpec((_B, 80, _ROWS, 128), lambda: (0, 0, 0, 0)),
            pl.BlockSpec((_B, 4, _ROWS, 128), lambda: (0, 0, 0, 0)),
            pl.BlockSpec((_ROWS, 128), lambda: (0, 0)),
            pl.BlockSpec((_ROWS, 128), lambda: (0, 0)),
            pl.BlockSpec((_ROWS, 128), lambda: (0, 0)),
        ],
        out_specs=pl.BlockSpec((_B, 8, _ROWS, 128), lambda: (0, 0, 0, 0)),
        out_shape=jax.ShapeDtypeStruct((_B, 8, _ROWS, 128), jnp.float32),
    )(cls, reg, jnp.asarray(_CX), jnp.asarray(_CY), jnp.asarray(_SV))


def _sc_nms(data):
    mesh = plsc.VectorSubcoreMesh(core_axis_name="c", subcore_axis_name="s")

    @functools.partial(
        pl.kernel,
        mesh=mesh,
        compiler_params=pltpu.CompilerParams(needs_layout_passes=False),
        out_type=jax.ShapeDtypeStruct((_B, _MAX_DET, 16), jnp.float32),
        scratch_types=[
            pltpu.VMEM((8, _NPAD), jnp.float32),
            pltpu.VMEM((7, _CLEN), jnp.float32),
            pltpu.VMEM((_MAX_DET, 16), jnp.float32),
        ],
    )
    def nms_kernel(data_hbm, out_hbm, data_v, comp_v, out_v):
        w = lax.axis_index("s") * 2 + lax.axis_index("c")

        @pl.when(w < _B)
        def _():
            pltpu.sync_copy(data_hbm.at[w], data_v)
            lane = lax.iota(jnp.int32, 16)
            zero16 = jnp.zeros((16,), jnp.float32)

            def zi(i, c):
                out_v[i] = zero16
                return c

            lax.fori_loop(0, _MAX_DET, zi, 0)

            NEG = jnp.float32(-3.4e38)
            BIGI = jnp.int32(2 ** 30)

            def cmp_chunk(i, off):
                base = i * 16
                s = data_v[0, pl.ds(base, 16)]
                msk = s > -1.0
                cum = plsc.cumsum(msk.astype(jnp.int32))
                pos = jnp.minimum(off + cum - 1, jnp.int32(_CAP + 15))
                plsc.store_scatter(comp_v, [jnp.zeros((16,), jnp.int32), pos],
                                   s, mask=msk)
                for r in range(1, 7):
                    v = data_v[r, pl.ds(base, 16)]
                    plsc.store_scatter(comp_v, [jnp.full((16,), r, jnp.int32), pos],
                                       v, mask=msk)
                npos = plsc.all_reduce_population_count(msk)[0]
                off = jnp.minimum(off + npos, jnp.int32(_CAP))
                return off

            k = lax.fori_loop(0, _NCHUNK, cmp_chunk, jnp.int32(0))
            plsc.store_scatter(comp_v, [jnp.zeros((16,), jnp.int32), k + lane],
                               jnp.full((16,), -2.0, jnp.float32))
            nch = (k + 15) // 16

            def amax_chunk(i, st):
                bv, bl = st
                v = comp_v[0, pl.ds(i * 16, 16)]
                liv = i * 16 + lane
                gt = v > bv
                bl = jnp.where(gt, liv, bl)
                bv = jnp.where(gt, v, bv)
                return bv, bl

            bv, bl = lax.fori_loop(
                0, nch, amax_chunk,
                (jnp.full((16,), NEG), jnp.zeros((16,), jnp.int32)))
            m0 = jnp.max(bv)
            idx0 = jnp.min(jnp.where(bv == m0, bl, BIGI))

            def step_body(st):
                cnt, m, idx = st
                rowv = jnp.minimum(lane, 6)
                idxv = jnp.full((16,), idx, jnp.int32)
                g = plsc.load_gather(comp_v, [rowv, idxv])
                bx1 = g[1]
                by1 = g[2]
                bx2 = g[3]
                by2 = g[4]
                ba = g[5]
                bk = g[6]
                row = jnp.where(lane == 0, bx1,
                      jnp.where(lane == 1, by1,
                      jnp.where(lane == 2, bx2,
                      jnp.where(lane == 3, by2,
                      jnp.where(lane == 4, bk,
                      jnp.where(lane == 5, m, 0.0))))))
                out_v[cnt] = row

                def sup_chunk(i, st2):
                    bv2, bl2 = st2
                    base = i * 16
                    s = comp_v[0, pl.ds(base, 16)]
                    x1 = comp_v[1, pl.ds(base, 16)]
                    y1 = comp_v[2, pl.ds(base, 16)]
                    x2 = comp_v[3, pl.ds(base, 16)]
                    y2 = comp_v[4, pl.ds(base, 16)]
                    ar = comp_v[5, pl.ds(base, 16)]
                    xx1 = jnp.maximum(bx1, x1)
                    yy1 = jnp.maximum(by1, y1)
                    xx2 = jnp.minimum(bx2, x2)
                    yy2 = jnp.minimum(by2, y2)
                    inter = jnp.maximum(xx2 - xx1, 0.0) * jnp.maximum(yy2 - yy1, 0.0)
                    iou = inter / (ba + ar - inter + 1e-9)
                    s = jnp.where(iou > _IOU_THR, -2.0, s)
                    comp_v[0, pl.ds(base, 16)] = s
                    liv = base + lane
                    gt = s > bv2
                    bl2 = jnp.where(gt, liv, bl2)
                    bv2 = jnp.where(gt, s, bv2)
                    return bv2, bl2

                bv2, bl2 = lax.fori_loop(
                    0, nch, sup_chunk,
                    (jnp.full((16,), NEG), jnp.zeros((16,), jnp.int32)))
                m2 = jnp.max(bv2)
                idx2 = jnp.min(jnp.where(bv2 == m2, bl2, BIGI))
                return cnt + 1, m2, idx2

            def step(i, st):
                return lax.cond(st[1] > _SCORE_THR, step_body, lambda s: s, st)

            lax.fori_loop(0, _MAX_DET, step, (jnp.int32(0), m0, idx0))
            pltpu.sync_copy(out_v, out_hbm.at[w])

    return nms_kernel(data)


def kernel(cls0, cls1, cls2, cnt0, cnt1, cnt2, reg0, reg1, reg2):
    del cnt0, cnt1, cnt2
    B = cls0.shape[0]
    cls = jnp.concatenate([c.reshape(B, 80, -1) for c in (cls0, cls1, cls2)], axis=2)
    reg = jnp.concatenate([r.reshape(B, 4, -1) for r in (reg0, reg1, reg2)], axis=2)
    cls = jnp.pad(cls, ((0, 0), (0, 0), (0, _NPAD - _N)), constant_values=-30.0)
    reg = jnp.pad(reg, ((0, 0), (0, 0), (0, _NPAD - _N)))
    cls = cls.reshape(B, 80, _ROWS, 128)
    reg = reg.reshape(B, 4, _ROWS, 128)

    data = _tc_decode(cls, reg).reshape(B, 8, _NPAD)
    out = _sc_nms(data)
    return out[:, :, :6]
